# baseline (speedup 1.0000x reference)
import functools
import jax
import jax.numpy as jnp
from jax import lax
from jax.experimental import pallas as pl
from jax.experimental.pallas import tpu as pltpu
from jax.experimental.pallas import tpu_sc as plsc

N = 10000
E = 320000
D_IN = 128
H1, C1 = 8, 8
NUM_CLASSES = 40
ROWS = 5000
CP = 128
H2C = 48
C1P = 80
NP = 10112


def _dense1_body(x_ref, w_ref, ams_ref, amd_ref, hcat_ref, ad_ref):
    h = jnp.dot(x_ref[...], w_ref[...], preferred_element_type=jnp.float32)
    a_s = jnp.dot(h, ams_ref[...], preferred_element_type=jnp.float32)
    z = jnp.zeros((h.shape[0], CP - H1 * C1 - 16), jnp.float32)
    hcat_ref[...] = jnp.concatenate([h, a_s, z], axis=1)
    ad_ref[...] = jnp.dot(h, amd_ref[...], preferred_element_type=jnp.float32)


def _dense1(x, W1i, ams, amd):
    return pl.pallas_call(
        _dense1_body,
        grid=(N // ROWS,),
        in_specs=[
            pl.BlockSpec((ROWS, D_IN), lambda i: (i, 0)),
            pl.BlockSpec((D_IN, H1 * C1), lambda i: (0, 0)),
            pl.BlockSpec((H1 * C1, 16), lambda i: (0, 0)),
            pl.BlockSpec((H1 * C1, 16), lambda i: (0, 0)),
        ],
        out_specs=[
            pl.BlockSpec((ROWS, CP), lambda i: (i, 0)),
            pl.BlockSpec((ROWS, 16), lambda i: (i, 0)),
        ],
        out_shape=[
            jax.ShapeDtypeStruct((N, CP), jnp.float32),
            jax.ShapeDtypeStruct((N, 16), jnp.float32),
        ],
    )(x, W1i, ams, amd)


def _dense2_body(acc_ref, e8_ref, b1_ref, w2_ref, ams_ref, amd_ref,
                 hcat_ref, ad_ref):
    acc = acc_ref[0] + acc_ref[1]
    num = acc[:, :H1 * C1]
    den = acc[:, H1 * C1:H1 * C1 + H1]
    den_exp = jnp.dot(den, e8_ref[...], preferred_element_type=jnp.float32)
    out1 = num / (den_exp + 1e-16) + b1_ref[...]
    t = jnp.where(out1 > 0, out1, jnp.exp(jnp.minimum(out1, 0.0)) - 1.0)
    h2 = jnp.dot(t, w2_ref[...], preferred_element_type=jnp.float32)
    a_s = jnp.dot(h2, ams_ref[...], preferred_element_type=jnp.float32)
    z = jnp.zeros((h2.shape[0], CP - H2C - 16), jnp.float32)
    hcat_ref[...] = jnp.concatenate([h2, a_s, z], axis=1)
    ad_ref[...] = jnp.dot(h2, amd_ref[...], preferred_element_type=jnp.float32)


def _dense2(acc1, e8, bias1, W2p, ams2, amd2):
    return pl.pallas_call(
        _dense2_body,
        grid=(N // ROWS,),
        in_specs=[
            pl.BlockSpec((2, ROWS, C1P), lambda i: (0, i, 0)),
            pl.BlockSpec((H1, H1 * C1), lambda i: (0, 0)),
            pl.BlockSpec((1, H1 * C1), lambda i: (0, 0)),
            pl.BlockSpec((H1 * C1, H2C), lambda i: (0, 0)),
            pl.BlockSpec((H2C, 16), lambda i: (0, 0)),
            pl.BlockSpec((H2C, 16), lambda i: (0, 0)),
        ],
        out_specs=[
            pl.BlockSpec((ROWS, CP), lambda i: (i, 0)),
            pl.BlockSpec((ROWS, 16), lambda i: (i, 0)),
        ],
        out_shape=[
            jax.ShapeDtypeStruct((N, CP), jnp.float32),
            jax.ShapeDtypeStruct((N, 16), jnp.float32),
        ],
    )(acc1, e8, bias1, W2p, ams2, amd2)


def _final_body(acc_ref, sel_ref, b2_ref, out_ref):
    acc = acc_ref[0] + acc_ref[1]
    den_exp = jnp.dot(acc, sel_ref[...], preferred_element_type=jnp.float32)
    out2 = acc[:, :NUM_CLASSES] / (den_exp + 1e-16) + b2_ref[...]
    m = jnp.max(out2, axis=1, keepdims=True)
    s = out2 - m
    out_ref[...] = s - jnp.log(jnp.sum(jnp.exp(s), axis=1, keepdims=True))


def _final(acc2, sel, bias2):
    return pl.pallas_call(
        _final_body,
        grid=(N // ROWS,),
        in_specs=[
            pl.BlockSpec((2, ROWS, H2C), lambda i: (0, i, 0)),
            pl.BlockSpec((H2C, NUM_CLASSES), lambda i: (0, 0)),
            pl.BlockSpec((1, NUM_CLASSES), lambda i: (0, 0)),
        ],
        out_specs=pl.BlockSpec((ROWS, NUM_CLASSES), lambda i: (i, 0)),
        out_shape=jax.ShapeDtypeStruct((N, NUM_CLASSES), jnp.float32),
    )(acc2, sel, bias2)


def _edge_pass_sc(src3, dst3, a_d, hcat, zeros, *, heads, ao, nh, chp):
    B = 100
    NW = 32
    EW = E // NW
    NCH = EW // B
    mesh = plsc.VectorSubcoreMesh(core_axis_name="c", subcore_axis_name="s")

    @functools.partial(
        pl.kernel,
        out_type=jax.ShapeDtypeStruct((2, NP, chp), jnp.float32),
        mesh=mesh,
        compiler_params=pltpu.CompilerParams(use_tc_tiling_on_sc=False),
        scratch_types=[
            pltpu.VMEM((NCH, B), jnp.int32),
            pltpu.VMEM((NCH, B), jnp.int32),
            pltpu.VMEM((2, B, 16), jnp.float32),
            pltpu.VMEM((2, B, CP), jnp.float32),
            pltpu.VMEM((2, B, chp), jnp.float32),
            pltpu.VMEM_SHARED((NP, chp), jnp.float32),
        ] + [pltpu.SemaphoreType.DMA] * 6,
    )
    def k(src_hbm, dst_hbm, ad_hbm, h_hbm, z_hbm, acc_hbm,
          src_big, dst_big, adr2, hr2, msg2, acc_sh, *sems):
        cid = lax.axis_index("c")
        sid = lax.axis_index("s")
        wid = cid * 16 + sid
        rpt = NP // 16
        pltpu.sync_copy(src_hbm.at[wid], src_big)
        pltpu.sync_copy(dst_hbm.at[wid], dst_big)
        pltpu.sync_copy(z_hbm, acc_sh.at[pl.ds(sid * rpt, rpt)])
        plsc.subcore_barrier()

        it = lax.iota(jnp.int32, 16)
        slots = [
            (adr2.at[j], hr2.at[j], msg2.at[j],
             sems[3 * j], sems[3 * j + 1], sems[3 * j + 2])
            for j in range(2)
        ]

        def gath_descs(ch, slot):
            adr_, hr_, _, sa, sh, _ = slot
            return (pltpu.make_async_copy(ad_hbm.at[dst_big.at[ch]], adr_, sa),
                    pltpu.make_async_copy(h_hbm.at[src_big.at[ch]], hr_, sh))

        def scat_desc(ch, slot):
            msg_, ss = slot[2], slot[5]
            return pltpu.make_async_copy(msg_, acc_sh.at[dst_big.at[ch]], ss)

        def compute(slot):
            adr_, hr_, msg_ = slot[0], slot[1], slot[2]
            for b in range(B):
                al = hr_[b, ao:ao + 16] + adr_[b, :]
                al = jnp.maximum(al, 0.0) + 0.2 * jnp.minimum(al, 0.0)
                exv = jnp.exp(al)
                for k2 in range(nh):
                    blk = hr_[b, 16 * k2:16 * (k2 + 1)] * exv
                    if heads == 1 and k2 == nh - 1:
                        blk = blk + jnp.where(it == 15, exv, 0.0)
                    msg_[b, 16 * k2:16 * (k2 + 1)] = blk
                if heads > 1:
                    msg_[b, 16 * nh:16 * (nh + 1)] = jnp.where(
                        it < heads, exv, 0.0)

        for j in range(2):
            for desc in gath_descs(j, slots[j]):
                desc.start()

        def body(i2, carry):
            for j in range(2):
                ch = 2 * i2 + j
                slot = slots[j]
                for desc in gath_descs(ch, slot):
                    desc.wait()

                @pl.when(i2 > 0)
                def _():
                    scat_desc(ch, slot).wait()

                compute(slot)
                pltpu.async_copy(slot[2], acc_sh.at[dst_big.at[ch]],
                                 slot[5], add=True)

                @pl.when(i2 < NCH // 2 - 1)
                def _():
                    for desc in gath_descs(ch + 2, slot):
                        desc.start()
            return carry

        lax.fori_loop(0, NCH // 2, body, 0)
        for j in range(2):
            scat_desc(j, slots[j]).wait()
        plsc.subcore_barrier()
        rs = sid * rpt
        pltpu.sync_copy(acc_sh.at[pl.ds(rs, rpt)],
                        acc_hbm.at[cid, pl.ds(rs, rpt)])

    return k(src3, dst3, a_d, hcat, zeros)


def kernel(x, edge_index, W1, att_src1, att_dst1, bias1, W2, att_src2, att_dst2, bias2):
    src3 = edge_index[0].reshape(32, (E // 32) // 100, 100)
    dst3 = edge_index[1].reshape(32, (E // 32) // 100, 100)

    rows64 = jnp.arange(H1 * C1)
    perm = (rows64 % H1) * C1 + rows64 // H1
    W1i = W1[:, perm]
    bias1i = bias1[perm]

    g64 = rows64 % H1
    c64 = rows64 // H1
    lane16 = jnp.arange(16)
    dupmask = ((lane16[None, :] % 8) == g64[:, None]).astype(jnp.float32)
    ams1 = att_src1[0, g64, c64][:, None] * dupmask
    amd1 = att_dst1[0, g64, c64][:, None] * dupmask

    e8 = (jnp.arange(H1)[:, None] == g64[None, :]).astype(jnp.float32)

    W2p = jnp.pad(W2[perm, :], ((0, 0), (0, H2C - NUM_CLASSES)))
    row48 = jnp.arange(H2C)
    keep2 = (row48 < NUM_CLASSES).astype(jnp.float32)
    a2pad = jnp.pad(jnp.ravel(att_src2), (0, H2C - NUM_CLASSES))
    d2pad = jnp.pad(jnp.ravel(att_dst2), (0, H2C - NUM_CLASSES))
    ams2 = jnp.broadcast_to((a2pad * keep2)[:, None], (H2C, 16))
    amd2 = jnp.broadcast_to((d2pad * keep2)[:, None], (H2C, 16))

    sel = jnp.broadcast_to(
        (jnp.arange(H2C) == H2C - 1).astype(jnp.float32)[:, None],
        (H2C, NUM_CLASSES))

    z1 = jnp.zeros((NP // 16, C1P), jnp.float32)
    z2 = jnp.zeros((NP // 16, H2C), jnp.float32)

    hcat1, a_d1 = _dense1(x, W1i, ams1, amd1)
    acc1 = _edge_pass_sc(src3, dst3, a_d1, hcat1, z1,
                         heads=H1, ao=64, nh=4, chp=C1P)
    hcat2, a_d2 = _dense2(acc1, e8, bias1i.reshape(1, -1), W2p, ams2, amd2)
    acc2 = _edge_pass_sc(src3, dst3, a_d2, hcat2, z2,
                         heads=1, ao=48, nh=3, chp=H2C)
    return _final(acc2, sel, bias2.reshape(1, -1))


# speedup vs baseline: 1.0010x; 1.0010x over previous
import functools
import jax
import jax.numpy as jnp
from jax import lax
from jax.experimental import pallas as pl
from jax.experimental.pallas import tpu as pltpu
from jax.experimental.pallas import tpu_sc as plsc

N = 10000
E = 320000
D_IN = 128
H1, C1 = 8, 8
NUM_CLASSES = 40
ROWS = 5000
CP = 128
H2C = 48
C1P = 80
NP = 10112


def _dense1_body(x_ref, w_ref, ams_ref, amd_ref, hcat_ref, ad_ref):
    h = jnp.dot(x_ref[...], w_ref[...], preferred_element_type=jnp.float32)
    a_s = jnp.dot(h, ams_ref[...], preferred_element_type=jnp.float32)
    z = jnp.zeros((h.shape[0], CP - H1 * C1 - 16), jnp.float32)
    hcat_ref[...] = jnp.concatenate([h, a_s, z], axis=1)
    ad_ref[...] = jnp.dot(h, amd_ref[...], preferred_element_type=jnp.float32)


def _dense1(x, W1i, ams, amd):
    return pl.pallas_call(
        _dense1_body,
        grid=(N // ROWS,),
        in_specs=[
            pl.BlockSpec((ROWS, D_IN), lambda i: (i, 0)),
            pl.BlockSpec((D_IN, H1 * C1), lambda i: (0, 0)),
            pl.BlockSpec((H1 * C1, 16), lambda i: (0, 0)),
            pl.BlockSpec((H1 * C1, 16), lambda i: (0, 0)),
        ],
        out_specs=[
            pl.BlockSpec((ROWS, CP), lambda i: (i, 0)),
            pl.BlockSpec((ROWS, 16), lambda i: (i, 0)),
        ],
        out_shape=[
            jax.ShapeDtypeStruct((N, CP), jnp.float32),
            jax.ShapeDtypeStruct((N, 16), jnp.float32),
        ],
    )(x, W1i, ams, amd)


def _dense2_body(acc_ref, e8_ref, b1_ref, w2_ref, ams_ref, amd_ref,
                 hcat_ref, ad_ref):
    acc = acc_ref[0] + acc_ref[1]
    num = acc[:, :H1 * C1]
    den = acc[:, H1 * C1:H1 * C1 + H1]
    den_exp = jnp.dot(den, e8_ref[...], preferred_element_type=jnp.float32)
    out1 = num / (den_exp + 1e-16) + b1_ref[...]
    t = jnp.where(out1 > 0, out1, jnp.exp(jnp.minimum(out1, 0.0)) - 1.0)
    h2 = jnp.dot(t, w2_ref[...], preferred_element_type=jnp.float32)
    a_s = jnp.dot(h2, ams_ref[...], preferred_element_type=jnp.float32)
    z = jnp.zeros((h2.shape[0], CP - H2C - 16), jnp.float32)
    hcat_ref[...] = jnp.concatenate([h2, a_s, z], axis=1)
    ad_ref[...] = jnp.dot(h2, amd_ref[...], preferred_element_type=jnp.float32)


def _dense2(acc1, e8, bias1, W2p, ams2, amd2):
    return pl.pallas_call(
        _dense2_body,
        grid=(N // ROWS,),
        in_specs=[
            pl.BlockSpec((2, ROWS, C1P), lambda i: (0, i, 0)),
            pl.BlockSpec((H1, H1 * C1), lambda i: (0, 0)),
            pl.BlockSpec((1, H1 * C1), lambda i: (0, 0)),
            pl.BlockSpec((H1 * C1, H2C), lambda i: (0, 0)),
            pl.BlockSpec((H2C, 16), lambda i: (0, 0)),
            pl.BlockSpec((H2C, 16), lambda i: (0, 0)),
        ],
        out_specs=[
            pl.BlockSpec((ROWS, CP), lambda i: (i, 0)),
            pl.BlockSpec((ROWS, 16), lambda i: (i, 0)),
        ],
        out_shape=[
            jax.ShapeDtypeStruct((N, CP), jnp.float32),
            jax.ShapeDtypeStruct((N, 16), jnp.float32),
        ],
    )(acc1, e8, bias1, W2p, ams2, amd2)


def _final_body(acc_ref, sel_ref, b2_ref, out_ref):
    acc = acc_ref[0] + acc_ref[1]
    den_exp = jnp.dot(acc, sel_ref[...], preferred_element_type=jnp.float32)
    out2 = acc[:, :NUM_CLASSES] / (den_exp + 1e-16) + b2_ref[...]
    m = jnp.max(out2, axis=1, keepdims=True)
    s = out2 - m
    out_ref[...] = s - jnp.log(jnp.sum(jnp.exp(s), axis=1, keepdims=True))


def _final(acc2, sel, bias2):
    return pl.pallas_call(
        _final_body,
        grid=(N // ROWS,),
        in_specs=[
            pl.BlockSpec((2, ROWS, H2C), lambda i: (0, i, 0)),
            pl.BlockSpec((H2C, NUM_CLASSES), lambda i: (0, 0)),
            pl.BlockSpec((1, NUM_CLASSES), lambda i: (0, 0)),
        ],
        out_specs=pl.BlockSpec((ROWS, NUM_CLASSES), lambda i: (i, 0)),
        out_shape=jax.ShapeDtypeStruct((N, NUM_CLASSES), jnp.float32),
    )(acc2, sel, bias2)


def _edge_pass_sc(src3, dst3, a_d, hcat, zeros, *, heads, ao, nh, chp):
    B = 100
    NW = 32
    EW = E // NW
    NCH = EW // B
    mesh = plsc.VectorSubcoreMesh(core_axis_name="c", subcore_axis_name="s")

    @functools.partial(
        pl.kernel,
        out_type=jax.ShapeDtypeStruct((2, NP, chp), jnp.float32),
        mesh=mesh,
        compiler_params=pltpu.CompilerParams(use_tc_tiling_on_sc=False),
        scratch_types=[
            pltpu.VMEM((NCH, B), jnp.int32),
            pltpu.VMEM((NCH, B), jnp.int32),
            pltpu.VMEM((2, B, 16), jnp.float32),
            pltpu.VMEM((2, B, CP), jnp.float32),
            pltpu.VMEM((2, B, chp), jnp.float32),
            pltpu.VMEM_SHARED((NP, chp), jnp.float32),
        ] + [pltpu.SemaphoreType.DMA] * 6,
    )
    def k(src_hbm, dst_hbm, ad_hbm, h_hbm, z_hbm, acc_hbm,
          src_big, dst_big, adr2, hr2, msg2, acc_sh, *sems):
        cid = lax.axis_index("c")
        sid = lax.axis_index("s")
        wid = cid * 16 + sid
        rpt = NP // 16
        pltpu.sync_copy(src_hbm.at[wid], src_big)
        pltpu.sync_copy(dst_hbm.at[wid], dst_big)
        pltpu.sync_copy(z_hbm, acc_sh.at[pl.ds(sid * rpt, rpt)])
        plsc.subcore_barrier()

        it = lax.iota(jnp.int32, 16)
        slots = [
            (adr2.at[j], hr2.at[j], msg2.at[j],
             sems[3 * j], sems[3 * j + 1], sems[3 * j + 2])
            for j in range(2)
        ]

        def gath_descs(ch, slot):
            adr_, hr_, _, sa, sh, _ = slot
            return (pltpu.make_async_copy(ad_hbm.at[dst_big.at[ch]], adr_, sa),
                    pltpu.make_async_copy(h_hbm.at[src_big.at[ch]], hr_, sh))

        def scat_desc(ch, slot):
            msg_, ss = slot[2], slot[5]
            return pltpu.make_async_copy(msg_, acc_sh.at[dst_big.at[ch]], ss)

        def compute(slot):
            adr_, hr_, msg_ = slot[0], slot[1], slot[2]
            for b in range(B):
                al = hr_[b, ao:ao + 16] + adr_[b, :]
                al = jnp.maximum(al, 0.2 * al)
                exv = jnp.exp(al)
                for k2 in range(nh):
                    blk = hr_[b, 16 * k2:16 * (k2 + 1)] * exv
                    if heads == 1 and k2 == nh - 1:
                        blk = blk + jnp.where(it == 15, exv, 0.0)
                    msg_[b, 16 * k2:16 * (k2 + 1)] = blk
                if heads > 1:
                    msg_[b, 16 * nh:16 * (nh + 1)] = exv

        for j in range(2):
            for desc in gath_descs(j, slots[j]):
                desc.start()

        def body(i2, carry):
            for j in range(2):
                ch = 2 * i2 + j
                slot = slots[j]
                for desc in gath_descs(ch, slot):
                    desc.wait()

                @pl.when(i2 > 0)
                def _():
                    scat_desc(ch, slot).wait()

                compute(slot)
                pltpu.async_copy(slot[2], acc_sh.at[dst_big.at[ch]],
                                 slot[5], add=True)

                @pl.when(i2 < NCH // 2 - 1)
                def _():
                    for desc in gath_descs(ch + 2, slot):
                        desc.start()
            return carry

        lax.fori_loop(0, NCH // 2, body, 0)
        for j in range(2):
            scat_desc(j, slots[j]).wait()
        plsc.subcore_barrier()
        rs = sid * rpt
        pltpu.sync_copy(acc_sh.at[pl.ds(rs, rpt)],
                        acc_hbm.at[cid, pl.ds(rs, rpt)])

    return k(src3, dst3, a_d, hcat, zeros)


def kernel(x, edge_index, W1, att_src1, att_dst1, bias1, W2, att_src2, att_dst2, bias2):
    src3 = edge_index[0].reshape(32, (E // 32) // 100, 100)
    dst3 = edge_index[1].reshape(32, (E // 32) // 100, 100)

    rows64 = jnp.arange(H1 * C1)
    perm = (rows64 % H1) * C1 + rows64 // H1
    W1i = W1[:, perm]
    bias1i = bias1[perm]

    g64 = rows64 % H1
    c64 = rows64 // H1
    lane16 = jnp.arange(16)
    dupmask = ((lane16[None, :] % 8) == g64[:, None]).astype(jnp.float32)
    ams1 = att_src1[0, g64, c64][:, None] * dupmask
    amd1 = att_dst1[0, g64, c64][:, None] * dupmask

    e8 = (jnp.arange(H1)[:, None] == g64[None, :]).astype(jnp.float32)

    W2p = jnp.pad(W2[perm, :], ((0, 0), (0, H2C - NUM_CLASSES)))
    row48 = jnp.arange(H2C)
    keep2 = (row48 < NUM_CLASSES).astype(jnp.float32)
    a2pad = jnp.pad(jnp.ravel(att_src2), (0, H2C - NUM_CLASSES))
    d2pad = jnp.pad(jnp.ravel(att_dst2), (0, H2C - NUM_CLASSES))
    ams2 = jnp.broadcast_to((a2pad * keep2)[:, None], (H2C, 16))
    amd2 = jnp.broadcast_to((d2pad * keep2)[:, None], (H2C, 16))

    sel = jnp.broadcast_to(
        (jnp.arange(H2C) == H2C - 1).astype(jnp.float32)[:, None],
        (H2C, NUM_CLASSES))

    z1 = jnp.zeros((NP // 16, C1P), jnp.float32)
    z2 = jnp.zeros((NP // 16, H2C), jnp.float32)

    hcat1, a_d1 = _dense1(x, W1i, ams1, amd1)
    acc1 = _edge_pass_sc(src3, dst3, a_d1, hcat1, z1,
                         heads=H1, ao=64, nh=4, chp=C1P)
    hcat2, a_d2 = _dense2(acc1, e8, bias1i.reshape(1, -1), W2p, ams2, amd2)
    acc2 = _edge_pass_sc(src3, dst3, a_d2, hcat2, z2,
                         heads=1, ao=48, nh=3, chp=H2C)
    return _final(acc2, sel, bias2.reshape(1, -1))


# speedup vs baseline: 1.1388x; 1.1376x over previous
import functools
import jax
import jax.numpy as jnp
from jax import lax
from jax.experimental import pallas as pl
from jax.experimental.pallas import tpu as pltpu
from jax.experimental.pallas import tpu_sc as plsc

N = 10000
E = 320000
D_IN = 128
H1, C1 = 8, 8
NUM_CLASSES = 40
ROWS = 5000
CP = 128
H2C = 48
C1P = 80
NP = 10112


def _dense1_body(x_ref, w_ref, ams_ref, amd_ref, hcat_ref, ad_ref):
    h = jnp.dot(x_ref[...], w_ref[...], preferred_element_type=jnp.float32)
    a_s = jnp.dot(h, ams_ref[...], preferred_element_type=jnp.float32)
    hcat_ref[...] = jnp.concatenate([h, a_s], axis=1)
    ad_ref[...] = jnp.dot(h, amd_ref[...], preferred_element_type=jnp.float32)


def _dense1(x, W1i, ams, amd):
    return pl.pallas_call(
        _dense1_body,
        grid=(N // ROWS,),
        in_specs=[
            pl.BlockSpec((ROWS, D_IN), lambda i: (i, 0)),
            pl.BlockSpec((D_IN, H1 * C1), lambda i: (0, 0)),
            pl.BlockSpec((H1 * C1, 16), lambda i: (0, 0)),
            pl.BlockSpec((H1 * C1, 16), lambda i: (0, 0)),
        ],
        out_specs=[
            pl.BlockSpec((ROWS, C1P), lambda i: (i, 0)),
            pl.BlockSpec((ROWS, 16), lambda i: (i, 0)),
        ],
        out_shape=[
            jax.ShapeDtypeStruct((N, C1P), jnp.float32),
            jax.ShapeDtypeStruct((N, 16), jnp.float32),
        ],
    )(x, W1i, ams, amd)


def _dense2_body(acc_ref, e8_ref, b1_ref, w2_ref, ams_ref, amd_ref,
                 hcat_ref, ad_ref):
    acc = acc_ref[0] + acc_ref[1]
    num = acc[:, :H1 * C1]
    den = acc[:, H1 * C1:H1 * C1 + H1]
    den_exp = jnp.dot(den, e8_ref[...], preferred_element_type=jnp.float32)
    out1 = num / (den_exp + 1e-16) + b1_ref[...]
    t = jnp.where(out1 > 0, out1, jnp.exp(jnp.minimum(out1, 0.0)) - 1.0)
    h2 = jnp.dot(t, w2_ref[...], preferred_element_type=jnp.float32)
    a_s = jnp.dot(h2, ams_ref[...], preferred_element_type=jnp.float32)
    hcat_ref[...] = jnp.concatenate([h2, a_s], axis=1)
    ad_ref[...] = jnp.dot(h2, amd_ref[...], preferred_element_type=jnp.float32)


def _dense2(acc1, e8, bias1, W2p, ams2, amd2):
    return pl.pallas_call(
        _dense2_body,
        grid=(N // ROWS,),
        in_specs=[
            pl.BlockSpec((2, ROWS, C1P), lambda i: (0, i, 0)),
            pl.BlockSpec((H1, H1 * C1), lambda i: (0, 0)),
            pl.BlockSpec((1, H1 * C1), lambda i: (0, 0)),
            pl.BlockSpec((H1 * C1, H2C), lambda i: (0, 0)),
            pl.BlockSpec((H2C, 16), lambda i: (0, 0)),
            pl.BlockSpec((H2C, 16), lambda i: (0, 0)),
        ],
        out_specs=[
            pl.BlockSpec((ROWS, H2C + 16), lambda i: (i, 0)),
            pl.BlockSpec((ROWS, 16), lambda i: (i, 0)),
        ],
        out_shape=[
            jax.ShapeDtypeStruct((N, H2C + 16), jnp.float32),
            jax.ShapeDtypeStruct((N, 16), jnp.float32),
        ],
    )(acc1, e8, bias1, W2p, ams2, amd2)


def _final_body(acc_ref, sel_ref, b2_ref, out_ref):
    acc = acc_ref[0] + acc_ref[1]
    den_exp = jnp.dot(acc, sel_ref[...], preferred_element_type=jnp.float32)
    out2 = acc[:, :NUM_CLASSES] / (den_exp + 1e-16) + b2_ref[...]
    m = jnp.max(out2, axis=1, keepdims=True)
    s = out2 - m
    out_ref[...] = s - jnp.log(jnp.sum(jnp.exp(s), axis=1, keepdims=True))


def _final(acc2, sel, bias2):
    return pl.pallas_call(
        _final_body,
        grid=(N // ROWS,),
        in_specs=[
            pl.BlockSpec((2, ROWS, H2C), lambda i: (0, i, 0)),
            pl.BlockSpec((H2C, NUM_CLASSES), lambda i: (0, 0)),
            pl.BlockSpec((1, NUM_CLASSES), lambda i: (0, 0)),
        ],
        out_specs=pl.BlockSpec((ROWS, NUM_CLASSES), lambda i: (i, 0)),
        out_shape=jax.ShapeDtypeStruct((N, NUM_CLASSES), jnp.float32),
    )(acc2, sel, bias2)


def _edge_pass_sc(src3, dst3, a_d, hcat, zeros, *, heads, ao, nh, chp):
    B = 100
    NW = 32
    EW = E // NW
    NCH = EW // B
    mesh = plsc.VectorSubcoreMesh(core_axis_name="c", subcore_axis_name="s")

    @functools.partial(
        pl.kernel,
        out_type=jax.ShapeDtypeStruct((2, NP, chp), jnp.float32),
        mesh=mesh,
        compiler_params=pltpu.CompilerParams(use_tc_tiling_on_sc=False),
        scratch_types=[
            pltpu.VMEM((NCH, B), jnp.int32),
            pltpu.VMEM((NCH, B), jnp.int32),
            pltpu.VMEM((2, B, 16), jnp.float32),
            pltpu.VMEM((2, B, ao + 16), jnp.float32),
            pltpu.VMEM((2, B, chp), jnp.float32),
            pltpu.VMEM_SHARED((NP, chp), jnp.float32),
        ] + [pltpu.SemaphoreType.DMA] * 6,
    )
    def k(src_hbm, dst_hbm, ad_hbm, h_hbm, z_hbm, acc_hbm,
          src_big, dst_big, adr2, hr2, msg2, acc_sh, *sems):
        cid = lax.axis_index("c")
        sid = lax.axis_index("s")
        wid = cid * 16 + sid
        rpt = NP // 16
        pltpu.sync_copy(src_hbm.at[wid], src_big)
        pltpu.sync_copy(dst_hbm.at[wid], dst_big)
        pltpu.sync_copy(z_hbm, acc_sh.at[pl.ds(sid * rpt, rpt)])
        plsc.subcore_barrier()

        it = lax.iota(jnp.int32, 16)
        slots = [
            (adr2.at[j], hr2.at[j], msg2.at[j],
             sems[3 * j], sems[3 * j + 1], sems[3 * j + 2])
            for j in range(2)
        ]

        def gath_descs(ch, slot):
            adr_, hr_, _, sa, sh, _ = slot
            return (pltpu.make_async_copy(ad_hbm.at[dst_big.at[ch]], adr_, sa),
                    pltpu.make_async_copy(h_hbm.at[src_big.at[ch]], hr_, sh))

        def scat_desc(ch, slot):
            msg_, ss = slot[2], slot[5]
            return pltpu.make_async_copy(msg_, acc_sh.at[dst_big.at[ch]], ss)

        def compute(slot):
            adr_, hr_, msg_ = slot[0], slot[1], slot[2]
            for b in range(B):
                al = hr_[b, ao:ao + 16] + adr_[b, :]
                al = jnp.maximum(al, 0.2 * al)
                exv = jnp.exp(al)
                for k2 in range(nh):
                    blk = hr_[b, 16 * k2:16 * (k2 + 1)] * exv
                    if heads == 1 and k2 == nh - 1:
                        blk = blk + jnp.where(it == 15, exv, 0.0)
                    msg_[b, 16 * k2:16 * (k2 + 1)] = blk
                if heads > 1:
                    msg_[b, 16 * nh:16 * (nh + 1)] = exv

        for j in range(2):
            for desc in gath_descs(j, slots[j]):
                desc.start()

        def body(i2, carry):
            for j in range(2):
                ch = 2 * i2 + j
                slot = slots[j]
                for desc in gath_descs(ch, slot):
                    desc.wait()

                @pl.when(i2 > 0)
                def _():
                    scat_desc(ch, slot).wait()

                compute(slot)
                pltpu.async_copy(slot[2], acc_sh.at[dst_big.at[ch]],
                                 slot[5], add=True)

                @pl.when(i2 < NCH // 2 - 1)
                def _():
                    for desc in gath_descs(ch + 2, slot):
                        desc.start()
            return carry

        lax.fori_loop(0, NCH // 2, body, 0)
        for j in range(2):
            scat_desc(j, slots[j]).wait()
        plsc.subcore_barrier()
        rs = sid * rpt
        pltpu.sync_copy(acc_sh.at[pl.ds(rs, rpt)],
                        acc_hbm.at[cid, pl.ds(rs, rpt)])

    return k(src3, dst3, a_d, hcat, zeros)


def kernel(x, edge_index, W1, att_src1, att_dst1, bias1, W2, att_src2, att_dst2, bias2):
    src3 = edge_index[0].reshape(32, (E // 32) // 100, 100)
    dst3 = edge_index[1].reshape(32, (E // 32) // 100, 100)

    rows64 = jnp.arange(H1 * C1)
    perm = (rows64 % H1) * C1 + rows64 // H1
    W1i = W1[:, perm]
    bias1i = bias1[perm]

    g64 = rows64 % H1
    c64 = rows64 // H1
    lane16 = jnp.arange(16)
    dupmask = ((lane16[None, :] % 8) == g64[:, None]).astype(jnp.float32)
    ams1 = att_src1[0, g64, c64][:, None] * dupmask
    amd1 = att_dst1[0, g64, c64][:, None] * dupmask

    e8 = (jnp.arange(H1)[:, None] == g64[None, :]).astype(jnp.float32)

    W2p = jnp.pad(W2[perm, :], ((0, 0), (0, H2C - NUM_CLASSES)))
    row48 = jnp.arange(H2C)
    keep2 = (row48 < NUM_CLASSES).astype(jnp.float32)
    a2pad = jnp.pad(jnp.ravel(att_src2), (0, H2C - NUM_CLASSES))
    d2pad = jnp.pad(jnp.ravel(att_dst2), (0, H2C - NUM_CLASSES))
    ams2 = jnp.broadcast_to((a2pad * keep2)[:, None], (H2C, 16))
    amd2 = jnp.broadcast_to((d2pad * keep2)[:, None], (H2C, 16))

    sel = jnp.broadcast_to(
        (jnp.arange(H2C) == H2C - 1).astype(jnp.float32)[:, None],
        (H2C, NUM_CLASSES))

    z1 = jnp.zeros((NP // 16, C1P), jnp.float32)
    z2 = jnp.zeros((NP // 16, H2C), jnp.float32)

    hcat1, a_d1 = _dense1(x, W1i, ams1, amd1)
    acc1 = _edge_pass_sc(src3, dst3, a_d1, hcat1, z1,
                         heads=H1, ao=64, nh=4, chp=C1P)
    hcat2, a_d2 = _dense2(acc1, e8, bias1i.reshape(1, -1), W2p, ams2, amd2)
    acc2 = _edge_pass_sc(src3, dst3, a_d2, hcat2, z2,
                         heads=1, ao=48, nh=3, chp=H2C)
    return _final(acc2, sel, bias2.reshape(1, -1))
